# Initial kernel scaffold; baseline (speedup 1.0000x reference)
import jax
import jax.numpy as jnp
from jax.experimental import pallas as pl


def kernel(x, ln1_w, ln2_w, Wq, Wk, Wv, Wo, gate_w, W1, W2):
    raise NotImplementedError("write your pallas kernel here")


# speedup vs baseline: 1.1896x; 1.1896x over previous
import jax
import jax.numpy as jnp
from jax.experimental import pallas as pl
from jax.experimental.pallas import tpu as pltpu

S, D, H, E, K, F = 2048, 768, 12, 8, 2, 3072
DH = D // H
BT = 256
NT = S // BT
NEG = -1e30


def _ln_qkv_body(x_ref, w_ref, ln_ref, qkv_ref):
    xf = x_ref[...]
    ms = jnp.mean(xf * xf, axis=-1, keepdims=True)
    xn = xf * jax.lax.rsqrt(ms + 1e-6) * ln_ref[...]
    qkv_ref[...] = jnp.dot(xn.astype(jnp.bfloat16), w_ref[...],
                           preferred_element_type=jnp.float32
                           ).astype(jnp.bfloat16)


def _ln_qkv(x2, wqkv_bf, ln1_w):
    return pl.pallas_call(
        _ln_qkv_body,
        grid=(NT,),
        in_specs=[
            pl.BlockSpec((BT, D), lambda i: (i, 0)),
            pl.BlockSpec((D, 3 * D), lambda i: (0, 0)),
            pl.BlockSpec((1, D), lambda i: (0, 0)),
        ],
        out_specs=pl.BlockSpec((BT, 3 * D), lambda i: (i, 0)),
        out_shape=jax.ShapeDtypeStruct((S, 3 * D), jnp.bfloat16),
    )(x2, wqkv_bf, ln1_w)


def _attn_body(qkv_ref, o_ref):
    i = pl.program_id(0)
    rows = i * BT + jax.lax.broadcasted_iota(jnp.int32, (BT, S), 0)
    cols = jax.lax.broadcasted_iota(jnp.int32, (BT, S), 1)
    causal = cols <= rows
    outs = []
    for h in range(H):
        q = qkv_ref[pl.ds(i * BT, BT), h * DH:(h + 1) * DH]
        k = qkv_ref[:, D + h * DH:D + (h + 1) * DH]
        v = qkv_ref[:, 2 * D + h * DH:2 * D + (h + 1) * DH]
        s = jax.lax.dot_general(q.astype(jnp.bfloat16),
                                k.astype(jnp.bfloat16),
                                (((1,), (1,)), ((), ())),
                                preferred_element_type=jnp.float32)
        s = s * (1.0 / 8.0)
        s = jnp.where(causal, s, NEG)
        m = jnp.max(s, axis=-1, keepdims=True)
        p = jnp.exp(s - m)
        p = p / jnp.sum(p, axis=-1, keepdims=True)
        outs.append(jnp.dot(p.astype(jnp.bfloat16), v.astype(jnp.bfloat16),
                            preferred_element_type=jnp.float32))
    o_ref[...] = jnp.concatenate(outs, axis=1)


def _attn(qkv):
    return pl.pallas_call(
        _attn_body,
        grid=(NT,),
        in_specs=[pl.BlockSpec((S, 3 * D), lambda i: (0, 0))],
        out_specs=pl.BlockSpec((BT, D), lambda i: (i, 0)),
        out_shape=jax.ShapeDtypeStruct((S, D), jnp.float32),
    )(qkv)


def _proj_route_body(o_ref, x_ref, wo_ref, ln_ref, gw_ref,
                     a_ref, h_ref, gates_ref):
    a = x_ref[...] + jnp.dot(o_ref[...].astype(jnp.bfloat16), wo_ref[...],
                             preferred_element_type=jnp.float32)
    a_ref[...] = a
    ms = jnp.mean(a * a, axis=-1, keepdims=True)
    hn = a * jax.lax.rsqrt(ms + 1e-6) * ln_ref[...]
    h_ref[...] = hn.astype(jnp.bfloat16)
    logits = jnp.dot(hn.astype(jnp.bfloat16), gw_ref[...],
                     preferred_element_type=jnp.float32)
    lane = jax.lax.broadcasted_iota(jnp.int32, (BT, 128), 1)
    logits = jnp.where(lane < E, logits, NEG)
    m1 = jnp.max(logits, axis=-1, keepdims=True)
    idx1 = jnp.min(jnp.where(logits == m1, lane, 127), axis=-1, keepdims=True)
    oh1 = (lane == idx1).astype(jnp.float32)
    lm = jnp.where(lane == idx1, NEG, logits)
    m2 = jnp.max(lm, axis=-1, keepdims=True)
    idx2 = jnp.min(jnp.where(lm == m2, lane, 127), axis=-1, keepdims=True)
    oh2 = (lane == idx2).astype(jnp.float32)
    d = jnp.exp(m2 - m1)
    p1 = 1.0 / (1.0 + d)
    p2 = d / (1.0 + d)
    gates_ref[...] = p1 * oh1 + p2 * oh2


def _proj_route(o, x2, wo_bf, ln2_w, gate_pad):
    return pl.pallas_call(
        _proj_route_body,
        grid=(NT,),
        in_specs=[
            pl.BlockSpec((BT, D), lambda i: (i, 0)),
            pl.BlockSpec((BT, D), lambda i: (i, 0)),
            pl.BlockSpec((D, D), lambda i: (0, 0)),
            pl.BlockSpec((1, D), lambda i: (0, 0)),
            pl.BlockSpec((D, 128), lambda i: (0, 0)),
        ],
        out_specs=[
            pl.BlockSpec((BT, D), lambda i: (i, 0)),
            pl.BlockSpec((BT, D), lambda i: (i, 0)),
            pl.BlockSpec((BT, 128), lambda i: (i, 0)),
        ],
        out_shape=[
            jax.ShapeDtypeStruct((S, D), jnp.float32),
            jax.ShapeDtypeStruct((S, D), jnp.bfloat16),
            jax.ShapeDtypeStruct((S, 128), jnp.float32),
        ],
    )(o, x2, wo_bf, ln2_w, gate_pad)


def _moe_body(h_ref, w1_ref, w2_ref, gates_ref, a_ref, out_ref, acc_ref):
    e = pl.program_id(0)
    i = pl.program_id(1)
    t = jnp.dot(h_ref[...], w1_ref[0], preferred_element_type=jnp.float32)
    act = t * jax.nn.sigmoid(t)
    y = jnp.dot(act.astype(jnp.bfloat16), w2_ref[0],
                preferred_element_type=jnp.float32)
    lane = jax.lax.broadcasted_iota(jnp.int32, (BT, 128), 1)
    g = jnp.sum(jnp.where(lane == e, gates_ref[...], 0.0),
                axis=-1, keepdims=True)
    val = g * y
    sl = pl.ds(i * BT, BT)

    @pl.when(e == 0)
    def _():
        acc_ref[sl, :] = val

    @pl.when(jnp.logical_and(e > 0, e < E - 1))
    def _():
        acc_ref[sl, :] += val

    @pl.when(e == E - 1)
    def _():
        out_ref[...] = a_ref[...] + acc_ref[sl, :] + val


def _moe(h_bf, w1_bf, w2_bf, gates, a):
    return pl.pallas_call(
        _moe_body,
        grid=(E, NT),
        in_specs=[
            pl.BlockSpec((BT, D), lambda e, i: (i, 0)),
            pl.BlockSpec((1, D, F), lambda e, i: (e, 0, 0)),
            pl.BlockSpec((1, F, D), lambda e, i: (e, 0, 0)),
            pl.BlockSpec((BT, 128), lambda e, i: (i, 0)),
            pl.BlockSpec((BT, D), lambda e, i: (i, 0)),
        ],
        out_specs=pl.BlockSpec((BT, D), lambda e, i: (i, 0)),
        out_shape=jax.ShapeDtypeStruct((S, D), jnp.float32),
        scratch_shapes=[pltpu.VMEM((S, D), jnp.float32)],
    )(h_bf, w1_bf, w2_bf, gates, a)


def kernel(x, ln1_w, ln2_w, Wq, Wk, Wv, Wo, gate_w, W1, W2):
    x2 = x.reshape(S, D)
    wqkv = jnp.concatenate([Wq, Wk, Wv], axis=1).astype(jnp.bfloat16)
    qkv = _ln_qkv(x2, wqkv, ln1_w.reshape(1, D))
    o = _attn(qkv)
    gate_pad = jnp.pad(gate_w, ((0, 0), (0, 128 - E))).astype(jnp.bfloat16)
    a, h_bf, gates = _proj_route(o, x2, Wo.astype(jnp.bfloat16),
                                 ln2_w.reshape(1, D), gate_pad)
    out = _moe(h_bf, W1.astype(jnp.bfloat16), W2.astype(jnp.bfloat16),
               gates, a)
    return out.reshape(1, S, D)
